# Initial kernel scaffold; baseline (speedup 1.0000x reference)
import jax
import jax.numpy as jnp
from jax.experimental import pallas as pl


def kernel(xyz, points, params):
    raise NotImplementedError("write your pallas kernel here")


# speedup vs baseline: 29.7645x; 29.7645x over previous
import functools

import jax
import jax.numpy as jnp
import numpy as np
from jax import lax
from jax.experimental import pallas as pl
from jax.experimental.pallas import tpu as pltpu
from jax.experimental.pallas import tpu_sc as plsc

B = 8
N = 2048
S = 512
IN_CH = 64
RADII = (0.1, 0.2, 0.4)
KS = (16, 32, 128)
C1S = (32, 64, 64)
C2S = (32, 64, 96)
C3S = (64, 128, 128)
CP = 128
R2 = tuple(np.float32(r ** 2) for r in RADII)

NC = 2
NSC = 16
NW = NC * NSC
RPW = (B * S) // NW

_F32 = jnp.float32
_I32 = jnp.int32


def _fps_kernel(xs_ref, ys_ref, zs_ref, nx_ref, ny_ref, nz_ref):
    xs = xs_ref[...]
    ys = ys_ref[...]
    zs = zs_ref[...]
    iota_n = lax.broadcasted_iota(_I32, (B, N), 1)
    iota_s = lax.broadcasted_iota(_I32, (B, S), 1)

    def body(i, carry):
        dist, far, ax, ay, az = carry
        onehot = iota_n == far
        cx = jnp.sum(jnp.where(onehot, xs, 0.0), axis=1, keepdims=True)
        cy = jnp.sum(jnp.where(onehot, ys, 0.0), axis=1, keepdims=True)
        cz = jnp.sum(jnp.where(onehot, zs, 0.0), axis=1, keepdims=True)
        dx = xs - cx
        dy = ys - cy
        dz = zs - cz
        d = (dx * dx + dy * dy) + dz * dz
        dist = jnp.minimum(dist, d)
        m = jnp.max(dist, axis=1, keepdims=True)
        far = jnp.min(jnp.where(dist == m, iota_n, N), axis=1, keepdims=True)
        sel = iota_s == i
        ax = jnp.where(sel, cx, ax)
        ay = jnp.where(sel, cy, ay)
        az = jnp.where(sel, cz, az)
        return dist, far, ax, ay, az

    init = (
        jnp.full((B, N), 1e10, _F32),
        jnp.zeros((B, 1), _I32),
        jnp.zeros((B, S), _F32),
        jnp.zeros((B, S), _F32),
        jnp.zeros((B, S), _F32),
    )
    _, _, ax, ay, az = lax.fori_loop(0, S, body, init)
    nx_ref[...] = ax
    ny_ref[...] = ay
    nz_ref[...] = az


def _fps_call(xs, ys, zs):
    return pl.pallas_call(
        _fps_kernel,
        out_shape=[jax.ShapeDtypeStruct((B, S), _F32)] * 3,
    )(xs, ys, zs)


def _dist_kernel(nxp_ref, xyzt_ref, out_ref):
    a = nxp_ref[0]
    bt = xyzt_ref[0]
    mm = jnp.dot(a, bt, preferred_element_type=_F32)
    ssrc = jnp.sum(a * a, axis=1, keepdims=True)
    sdst = jnp.sum(bt * bt, axis=0, keepdims=True)
    out_ref[0] = (-2.0 * mm + ssrc) + sdst


def _dist_call(nxp, xyztp):
    return pl.pallas_call(
        _dist_kernel,
        grid=(B,),
        in_specs=[
            pl.BlockSpec((1, S, 8), lambda b: (b, 0, 0)),
            pl.BlockSpec((1, 8, N), lambda b: (b, 0, 0)),
        ],
        out_specs=pl.BlockSpec((1, S, N), lambda b: (b, 0, 0)),
        out_shape=jax.ShapeDtypeStruct((B, S, N), _F32),
    )(nxp, xyztp)


def _p1_kernel(pts_ref, xp_ref, wp1, wx1, wp2, wx2, wp3, wx3, t1, t2, t3):
    p = pts_ref[...]
    x = xp_ref[...]
    t1[...] = (jnp.dot(p, wp1[...], preferred_element_type=_F32)
               + jnp.dot(x, wx1[...], preferred_element_type=_F32))
    t2[...] = (jnp.dot(p, wp2[...], preferred_element_type=_F32)
               + jnp.dot(x, wx2[...], preferred_element_type=_F32))
    t3[...] = (jnp.dot(p, wp3[...], preferred_element_type=_F32)
               + jnp.dot(x, wx3[...], preferred_element_type=_F32))


def _p1_call(pts2d, xp2d, wps, wxs):
    rb = 2048
    grid = (B * N // rb,)
    wspec = lambda shp: pl.BlockSpec(shp, lambda i: (0, 0))
    return pl.pallas_call(
        _p1_kernel,
        grid=grid,
        in_specs=[
            pl.BlockSpec((rb, IN_CH), lambda i: (i, 0)),
            pl.BlockSpec((rb, 8), lambda i: (i, 0)),
            wspec((IN_CH, CP)), wspec((8, CP)),
            wspec((IN_CH, CP)), wspec((8, CP)),
            wspec((IN_CH, CP)), wspec((8, CP)),
        ],
        out_specs=[
            pl.BlockSpec((rb, CP), lambda i: (i, 0)),
            pl.BlockSpec((rb, CP), lambda i: (i, 0)),
            pl.BlockSpec((rb, CP), lambda i: (i, 0)),
        ],
        out_shape=[
            jax.ShapeDtypeStruct((B * N, CP), _F32),
            jax.ShapeDtypeStruct((B * N, CP), _F32),
            jax.ShapeDtypeStruct((B * N, CP), _F32),
        ],
    )(pts2d, xp2d, wps[0], wxs[0], wps[1], wxs[1], wps[2], wxs[2])


def _sc_kernel(dist_hbm, t1_hbm, t2_hbm, t3_hbm,
               g1_hbm, g2_hbm, g3_hbm, cnt_hbm,
               dv, sb1, sb2, sb3, gb1, gb2, gb3, cbuf,
               r1v, r2v, r3v, sem1, sem2, sem3):
    cid = lax.axis_index("c")
    sid = lax.axis_index("s")
    wid = sid * NC + cid
    row0 = wid * RPW
    boff = (row0 // S) * N

    for j in range((KS[0] + 32) // 16):
        sb1[pl.ds(j * 16, 16)] = lax.iota(_I32, 16) + j * 16
    for j in range((KS[1] + 32) // 16):
        sb2[pl.ds(j * 16, 16)] = lax.iota(_I32, 16) + j * 16
    for j in range((KS[2] + 32) // 16):
        sb3[pl.ds(j * 16, 16)] = lax.iota(_I32, 16) + j * 16

    def row_body(r, carry):
        row = row0 + r
        pltpu.sync_copy(dist_hbm.at[row], dv)

        def chunk(c, tc):
            t1, t2, t3 = tc
            d = dv[pl.ds(c * 16, 16)]
            idxv = lax.iota(_I32, 16) + c * 16
            m1 = d <= R2[0]
            m2 = d <= R2[1]
            m3 = d <= R2[2]
            plsc.store_compressed(
                sb3.at[pl.ds(jnp.minimum(t3, KS[2]), 16)], idxv, mask=m3)
            plsc.store_compressed(
                sb2.at[pl.ds(jnp.minimum(t2, KS[1]), 16)], idxv, mask=m2)
            plsc.store_compressed(
                sb1.at[pl.ds(jnp.minimum(t1, KS[0]), 16)], idxv, mask=m1)
            t1 = t1 + jnp.sum(m1.astype(_I32))
            t2 = t2 + jnp.sum(m2.astype(_I32))
            t3 = t3 + jnp.sum(m3.astype(_I32))
            return t1, t2, t3

        zero = jnp.zeros((), _I32)
        t1, t2, t3 = lax.fori_loop(0, N // 16, chunk, (zero, zero, zero))

        lane = lax.iota(_I32, 16)
        cbuf[...] = ((lane == 0).astype(_I32) * t1
                     + (lane == 1).astype(_I32) * t2
                     + (lane == 2).astype(_I32) * t3)
        pltpu.sync_copy(cbuf, cnt_hbm.at[row])

        for j in range(KS[0] // 16):
            gb1[pl.ds(j * 16, 16)] = sb1[pl.ds(j * 16, 16)] + boff
        for j in range(KS[1] // 16):
            gb2[pl.ds(j * 16, 16)] = sb2[pl.ds(j * 16, 16)] + boff
        for j in range(KS[2] // 16):
            gb3[pl.ds(j * 16, 16)] = sb3[pl.ds(j * 16, 16)] + boff

        cp1 = pltpu.async_copy(t1_hbm.at[gb1], r1v, sem1)
        cp2 = pltpu.async_copy(t2_hbm.at[gb2], r2v, sem2)
        cp3 = pltpu.async_copy(t3_hbm.at[gb3], r3v, sem3)
        cp1.wait()
        pltpu.sync_copy(r1v, g1_hbm.at[row])
        cp2.wait()
        pltpu.sync_copy(r2v, g2_hbm.at[row])
        cp3.wait()
        pltpu.sync_copy(r3v, g3_hbm.at[row])
        return carry

    lax.fori_loop(0, RPW, row_body, jnp.zeros((), _I32))


def _sc_call(dist2d, t1, t2, t3):
    mesh = plsc.VectorSubcoreMesh(core_axis_name="c", subcore_axis_name="s",
                                  num_cores=NC, num_subcores=NSC)
    fn = functools.partial(
        pl.kernel,
        out_type=[
            jax.ShapeDtypeStruct((B * S, KS[0], CP), _F32),
            jax.ShapeDtypeStruct((B * S, KS[1], CP), _F32),
            jax.ShapeDtypeStruct((B * S, KS[2], CP), _F32),
            jax.ShapeDtypeStruct((B * S, 16), _I32),
        ],
        mesh=mesh,
        compiler_params=pltpu.CompilerParams(needs_layout_passes=False),
        scratch_types=[
            pltpu.VMEM((N,), _F32),
            pltpu.VMEM((KS[0] + 32,), _I32),
            pltpu.VMEM((KS[1] + 32,), _I32),
            pltpu.VMEM((KS[2] + 32,), _I32),
            pltpu.VMEM((KS[0],), _I32),
            pltpu.VMEM((KS[1],), _I32),
            pltpu.VMEM((KS[2],), _I32),
            pltpu.VMEM((16,), _I32),
            pltpu.VMEM((KS[0], CP), _F32),
            pltpu.VMEM((KS[1], CP), _F32),
            pltpu.VMEM((KS[2], CP), _F32),
            pltpu.SemaphoreType.DMA,
            pltpu.SemaphoreType.DMA,
            pltpu.SemaphoreType.DMA,
        ],
    )(_sc_kernel)
    return fn(dist2d, t1, t2, t3)


def _make_mlp_kernel(sb, k, scale):
    def _mlp_kernel(g_ref, nx_ref, cnt_ref, wx, b1, w2, b2, w3, b3, o_ref):
        nx = nx_ref[...]
        cterm = jnp.dot(nx, wx[...], preferred_element_type=_F32) - b1[...]
        g = g_ref[...]
        h1 = jnp.maximum(g - cterm[:, None, :], 0.0)
        h1 = h1.reshape(sb * k, CP)
        h2 = jnp.maximum(
            jnp.dot(h1, w2[...], preferred_element_type=_F32) + b2[...], 0.0)
        h3 = jnp.maximum(
            jnp.dot(h2, w3[...], preferred_element_type=_F32) + b3[...], 0.0)
        h3 = h3.reshape(sb, k, CP)
        t = cnt_ref[...][:, scale:scale + 1]
        kio = lax.broadcasted_iota(_I32, (sb, k, CP), 1)
        t3 = lax.broadcast_in_dim(t, (sb, k, CP), (0, 1))
        valid = kio < t3
        h3 = jnp.where(valid, h3, 0.0)
        o_ref[...] = jnp.max(h3, axis=1)

    return _mlp_kernel


def _mlp_call(scale, g, nxp2d, cnt, wx, b1, w2, b2, w3, b3):
    k = KS[scale]
    sb = {0: 256, 1: 256, 2: 64}[scale]
    grid = (B * S // sb,)
    wspec = lambda shp: pl.BlockSpec(shp, lambda i: (0, 0))
    return pl.pallas_call(
        _make_mlp_kernel(sb, k, scale),
        grid=grid,
        in_specs=[
            pl.BlockSpec((sb, k, CP), lambda i: (i, 0, 0)),
            pl.BlockSpec((sb, 8), lambda i: (i, 0)),
            pl.BlockSpec((sb, 16), lambda i: (i, 0)),
            wspec((8, CP)),
            wspec((1, CP)),
            wspec((CP, CP)),
            wspec((1, CP)),
            wspec((CP, CP)),
            wspec((1, CP)),
        ],
        out_specs=pl.BlockSpec((sb, CP), lambda i: (i, 0)),
        out_shape=jax.ShapeDtypeStruct((B * S, CP), _F32),
    )(g, nxp2d, cnt, wx, b1, w2, b2, w3, b3)


def _fold_bn(layer):
    a = layer['gamma'] / jnp.sqrt(layer['rv'] + 1e-5)
    w = layer['W'] * a[:, None]
    b = layer['b'] * a + layer['beta'] - layer['rm'] * a
    return w, b


def kernel(xyz, points, params):
    xs = xyz[:, :, 0]
    ys = xyz[:, :, 1]
    zs = xyz[:, :, 2]
    nx, ny, nz = _fps_call(xs, ys, zs)
    new_xyz = jnp.stack([nx, ny, nz], axis=-1)

    nxp = jnp.concatenate([new_xyz, jnp.zeros((B, S, 5), _F32)], axis=-1)
    xyzt = jnp.transpose(xyz, (0, 2, 1))
    xyztp = jnp.concatenate([xyzt, jnp.zeros((B, 5, N), _F32)], axis=1)
    dist = _dist_call(nxp, xyztp)

    folded = [[_fold_bn(l) for l in params[i]] for i in range(3)]

    def _padc(a):
        return jnp.concatenate(
            [a, jnp.zeros(a.shape[:-1] + (CP - a.shape[-1],), _F32)], -1)

    wps, wxs, b1s = [], [], []
    for i in range(3):
        w1, b1 = folded[i][0]
        wps.append(_padc(jnp.transpose(w1[:, :IN_CH])))
        wx = jnp.transpose(w1[:, IN_CH:])
        wx = jnp.concatenate([wx, jnp.zeros((5, C1S[i]), _F32)], 0)
        wxs.append(_padc(wx))
        b1s.append(_padc(b1[None, :]))

    pts2d = points.reshape(B * N, IN_CH)
    xp2d = jnp.concatenate(
        [xyz.reshape(B * N, 3), jnp.zeros((B * N, 5), _F32)], axis=-1)
    t1, t2, t3 = _p1_call(pts2d, xp2d, wps, wxs)

    g1, g2, g3, cnt = _sc_call(dist.reshape(B * S, N), t1, t2, t3)

    nxp2d = nxp.reshape(B * S, 8)
    outs = []
    for i, g in enumerate((g1, g2, g3)):
        w2, b2 = folded[i][1]
        w3, b3 = folded[i][2]
        w2p = jnp.zeros((CP, CP), _F32).at[:C1S[i], :C2S[i]].set(
            jnp.transpose(w2))
        w3p = jnp.zeros((CP, CP), _F32).at[:C2S[i], :C3S[i]].set(
            jnp.transpose(w3))
        outs.append(_mlp_call(
            i, g, nxp2d, cnt, wxs[i], b1s[i],
            w2p, _padc(b2[None, :]),
            w3p, _padc(b3[None, :])))

    new_points = jnp.concatenate(
        [outs[0][:, :C3S[0]].reshape(B, S, C3S[0]),
         outs[1][:, :C3S[1]].reshape(B, S, C3S[1]),
         outs[2][:, :C3S[2]].reshape(B, S, C3S[2])], axis=-1)
    return (new_xyz, new_points)


# speedup vs baseline: 31.7386x; 1.0663x over previous
import functools

import jax
import jax.numpy as jnp
import numpy as np
from jax import lax
from jax.experimental import pallas as pl
from jax.experimental.pallas import tpu as pltpu
from jax.experimental.pallas import tpu_sc as plsc

B = 8
N = 2048
S = 512
IN_CH = 64
RADII = (0.1, 0.2, 0.4)
KS = (16, 32, 128)
C1S = (32, 64, 64)
C2S = (32, 64, 96)
C3S = (64, 128, 128)
CP = 128
R2 = tuple(np.float32(r ** 2) for r in RADII)

NC = 2
NSC = 16
NW = NC * NSC
RPW = (B * S) // NW

_F32 = jnp.float32
_I32 = jnp.int32


def _fps_kernel(xs_ref, ys_ref, zs_ref, nx_ref, ny_ref, nz_ref):
    xs = xs_ref[...]
    ys = ys_ref[...]
    zs = zs_ref[...]
    iota_n = lax.broadcasted_iota(_I32, (B, N), 1)
    iota_s = lax.broadcasted_iota(_I32, (B, S), 1)

    def body(i, carry):
        dist, far, ax, ay, az = carry
        onehot = iota_n == far
        cx = jnp.sum(jnp.where(onehot, xs, 0.0), axis=1, keepdims=True)
        cy = jnp.sum(jnp.where(onehot, ys, 0.0), axis=1, keepdims=True)
        cz = jnp.sum(jnp.where(onehot, zs, 0.0), axis=1, keepdims=True)
        dx = xs - cx
        dy = ys - cy
        dz = zs - cz
        d = (dx * dx + dy * dy) + dz * dz
        dist = jnp.minimum(dist, d)
        m = jnp.max(dist, axis=1, keepdims=True)
        far = jnp.min(jnp.where(dist == m, iota_n, N), axis=1, keepdims=True)
        sel = iota_s == i
        ax = jnp.where(sel, cx, ax)
        ay = jnp.where(sel, cy, ay)
        az = jnp.where(sel, cz, az)
        return dist, far, ax, ay, az

    init = (
        jnp.full((B, N), 1e10, _F32),
        jnp.zeros((B, 1), _I32),
        jnp.zeros((B, S), _F32),
        jnp.zeros((B, S), _F32),
        jnp.zeros((B, S), _F32),
    )
    _, _, ax, ay, az = lax.fori_loop(0, S, body, init)
    nx_ref[...] = ax
    ny_ref[...] = ay
    nz_ref[...] = az


def _fps_call(xs, ys, zs):
    return pl.pallas_call(
        _fps_kernel,
        out_shape=[jax.ShapeDtypeStruct((B, S), _F32)] * 3,
    )(xs, ys, zs)


def _dist_kernel(nxp_ref, xyzt_ref, out_ref):
    a = nxp_ref[0]
    bt = xyzt_ref[0]
    mm = jnp.dot(a, bt, preferred_element_type=_F32)
    ssrc = jnp.sum(a * a, axis=1, keepdims=True)
    sdst = jnp.sum(bt * bt, axis=0, keepdims=True)
    out_ref[0] = (-2.0 * mm + ssrc) + sdst


def _dist_call(nxp, xyztp):
    return pl.pallas_call(
        _dist_kernel,
        grid=(B,),
        in_specs=[
            pl.BlockSpec((1, S, 8), lambda b: (b, 0, 0)),
            pl.BlockSpec((1, 8, N), lambda b: (b, 0, 0)),
        ],
        out_specs=pl.BlockSpec((1, S, N), lambda b: (b, 0, 0)),
        out_shape=jax.ShapeDtypeStruct((B, S, N), _F32),
    )(nxp, xyztp)


def _p1_kernel(pts_ref, xp_ref, wp1, wx1, wp2, wx2, wp3, wx3, t1, t2, t3):
    p = pts_ref[...]
    x = xp_ref[...]
    t1[...] = (jnp.dot(p, wp1[...], preferred_element_type=_F32)
               + jnp.dot(x, wx1[...], preferred_element_type=_F32))
    t2[...] = (jnp.dot(p, wp2[...], preferred_element_type=_F32)
               + jnp.dot(x, wx2[...], preferred_element_type=_F32))
    t3[...] = (jnp.dot(p, wp3[...], preferred_element_type=_F32)
               + jnp.dot(x, wx3[...], preferred_element_type=_F32))


def _p1_call(pts2d, xp2d, wps, wxs):
    rb = 2048
    grid = (B * N // rb,)
    wspec = lambda shp: pl.BlockSpec(shp, lambda i: (0, 0))
    return pl.pallas_call(
        _p1_kernel,
        grid=grid,
        in_specs=[
            pl.BlockSpec((rb, IN_CH), lambda i: (i, 0)),
            pl.BlockSpec((rb, 8), lambda i: (i, 0)),
            wspec((IN_CH, CP)), wspec((8, CP)),
            wspec((IN_CH, CP)), wspec((8, CP)),
            wspec((IN_CH, CP)), wspec((8, CP)),
        ],
        out_specs=[
            pl.BlockSpec((rb, CP), lambda i: (i, 0)),
            pl.BlockSpec((rb, CP), lambda i: (i, 0)),
            pl.BlockSpec((rb, CP), lambda i: (i, 0)),
        ],
        out_shape=[
            jax.ShapeDtypeStruct((B * N, CP), _F32),
            jax.ShapeDtypeStruct((B * N, CP), _F32),
            jax.ShapeDtypeStruct((B * N, CP), _F32),
        ],
    )(pts2d, xp2d, wps[0], wxs[0], wps[1], wxs[1], wps[2], wxs[2])


def _sc_kernel(dist_hbm, t1_hbm, t2_hbm, t3_hbm,
               g1_hbm, g2_hbm, g3_hbm, cnt_hbm,
               dv, sb1, sb2, sb3, gb1, gb2, gb3, gb1b, gb2b, gb3b, cbuf,
               r1v, r2v, r3v, sem1, sem2, sem3):
    cid = lax.axis_index("c")
    sid = lax.axis_index("s")
    wid = sid * NC + cid
    row0 = wid * RPW
    boff = (row0 // S) * N

    for j in range((KS[0] + 32) // 16):
        sb1[pl.ds(j * 16, 16)] = lax.iota(_I32, 16) + j * 16
    for j in range((KS[1] + 32) // 16):
        sb2[pl.ds(j * 16, 16)] = lax.iota(_I32, 16) + j * 16
    for j in range((KS[2] + 32) // 16):
        sb3[pl.ds(j * 16, 16)] = lax.iota(_I32, 16) + j * 16

    def _select_row(row, gb1x, gb2x, gb3x):
        pltpu.sync_copy(dist_hbm.at[row], dv)

        def chunk(c, tc):
            t1, t2, t3 = tc
            d = dv[pl.ds(c * 16, 16)]
            idxv = lax.iota(_I32, 16) + c * 16
            m1 = d <= R2[0]
            m2 = d <= R2[1]
            m3 = d <= R2[2]
            plsc.store_compressed(
                sb3.at[pl.ds(jnp.minimum(t3, KS[2]), 16)], idxv, mask=m3)
            plsc.store_compressed(
                sb2.at[pl.ds(jnp.minimum(t2, KS[1]), 16)], idxv, mask=m2)
            plsc.store_compressed(
                sb1.at[pl.ds(jnp.minimum(t1, KS[0]), 16)], idxv, mask=m1)
            t1 = t1 + jnp.sum(m1.astype(_I32))
            t2 = t2 + jnp.sum(m2.astype(_I32))
            t3 = t3 + jnp.sum(m3.astype(_I32))
            return t1, t2, t3

        zero = jnp.zeros((), _I32)
        t1, t2, t3 = lax.fori_loop(0, N // 16, chunk, (zero, zero, zero))

        lane = lax.iota(_I32, 16)
        cbuf[...] = ((lane == 0).astype(_I32) * t1
                     + (lane == 1).astype(_I32) * t2
                     + (lane == 2).astype(_I32) * t3)
        pltpu.sync_copy(cbuf, cnt_hbm.at[row])

        for j in range(KS[0] // 16):
            gb1x[pl.ds(j * 16, 16)] = sb1[pl.ds(j * 16, 16)] + boff
        for j in range(KS[1] // 16):
            gb2x[pl.ds(j * 16, 16)] = sb2[pl.ds(j * 16, 16)] + boff
        for j in range(KS[2] // 16):
            gb3x[pl.ds(j * 16, 16)] = sb3[pl.ds(j * 16, 16)] + boff

    def pair_body(p, carry):
        ra = row0 + 2 * p
        rb = ra + 1
        _select_row(ra, gb1, gb2, gb3)
        cp1 = pltpu.async_copy(t1_hbm.at[gb1], r1v, sem1)
        cp2 = pltpu.async_copy(t2_hbm.at[gb2], r2v, sem2)
        cp3 = pltpu.async_copy(t3_hbm.at[gb3], r3v, sem3)
        _select_row(rb, gb1b, gb2b, gb3b)
        cp1.wait()
        pltpu.sync_copy(r1v, g1_hbm.at[ra])
        cp2.wait()
        pltpu.sync_copy(r2v, g2_hbm.at[ra])
        cp3.wait()
        pltpu.sync_copy(r3v, g3_hbm.at[ra])
        dp1 = pltpu.async_copy(t1_hbm.at[gb1b], r1v, sem1)
        dp2 = pltpu.async_copy(t2_hbm.at[gb2b], r2v, sem2)
        dp3 = pltpu.async_copy(t3_hbm.at[gb3b], r3v, sem3)
        dp1.wait()
        pltpu.sync_copy(r1v, g1_hbm.at[rb])
        dp2.wait()
        pltpu.sync_copy(r2v, g2_hbm.at[rb])
        dp3.wait()
        pltpu.sync_copy(r3v, g3_hbm.at[rb])
        return carry

    lax.fori_loop(0, RPW // 2, pair_body, jnp.zeros((), _I32))


def _sc_call(dist2d, t1, t2, t3):
    mesh = plsc.VectorSubcoreMesh(core_axis_name="c", subcore_axis_name="s",
                                  num_cores=NC, num_subcores=NSC)
    fn = functools.partial(
        pl.kernel,
        out_type=[
            jax.ShapeDtypeStruct((B * S, KS[0], CP), _F32),
            jax.ShapeDtypeStruct((B * S, KS[1], CP), _F32),
            jax.ShapeDtypeStruct((B * S, KS[2], CP), _F32),
            jax.ShapeDtypeStruct((B * S, 16), _I32),
        ],
        mesh=mesh,
        compiler_params=pltpu.CompilerParams(needs_layout_passes=False),
        scratch_types=[
            pltpu.VMEM((N,), _F32),
            pltpu.VMEM((KS[0] + 32,), _I32),
            pltpu.VMEM((KS[1] + 32,), _I32),
            pltpu.VMEM((KS[2] + 32,), _I32),
            pltpu.VMEM((KS[0],), _I32),
            pltpu.VMEM((KS[1],), _I32),
            pltpu.VMEM((KS[2],), _I32),
            pltpu.VMEM((KS[0],), _I32),
            pltpu.VMEM((KS[1],), _I32),
            pltpu.VMEM((KS[2],), _I32),
            pltpu.VMEM((16,), _I32),
            pltpu.VMEM((KS[0], CP), _F32),
            pltpu.VMEM((KS[1], CP), _F32),
            pltpu.VMEM((KS[2], CP), _F32),
            pltpu.SemaphoreType.DMA,
            pltpu.SemaphoreType.DMA,
            pltpu.SemaphoreType.DMA,
        ],
    )(_sc_kernel)
    return fn(dist2d, t1, t2, t3)


def _make_mlp_kernel(sb, k, scale):
    def _mlp_kernel(g_ref, nx_ref, cnt_ref, wx, b1, w2, b2, w3, b3, o_ref):
        nx = nx_ref[...]
        cterm = jnp.dot(nx, wx[...], preferred_element_type=_F32) - b1[...]
        g = g_ref[...]
        h1 = jnp.maximum(g - cterm[:, None, :], 0.0)
        h1 = h1.reshape(sb * k, CP)
        h2 = jnp.maximum(
            jnp.dot(h1, w2[...], preferred_element_type=_F32) + b2[...], 0.0)
        h3 = jnp.maximum(
            jnp.dot(h2, w3[...], preferred_element_type=_F32) + b3[...], 0.0)
        h3 = h3.reshape(sb, k, CP)
        t = cnt_ref[...][:, scale:scale + 1]
        kio = lax.broadcasted_iota(_I32, (sb, k, CP), 1)
        t3 = lax.broadcast_in_dim(t, (sb, k, CP), (0, 1))
        valid = kio < t3
        h3 = jnp.where(valid, h3, 0.0)
        o_ref[...] = jnp.max(h3, axis=1)

    return _mlp_kernel


def _mlp_call(scale, g, nxp2d, cnt, wx, b1, w2, b2, w3, b3):
    k = KS[scale]
    sb = {0: 256, 1: 256, 2: 64}[scale]
    grid = (B * S // sb,)
    wspec = lambda shp: pl.BlockSpec(shp, lambda i: (0, 0))
    return pl.pallas_call(
        _make_mlp_kernel(sb, k, scale),
        grid=grid,
        in_specs=[
            pl.BlockSpec((sb, k, CP), lambda i: (i, 0, 0)),
            pl.BlockSpec((sb, 8), lambda i: (i, 0)),
            pl.BlockSpec((sb, 16), lambda i: (i, 0)),
            wspec((8, CP)),
            wspec((1, CP)),
            wspec((CP, CP)),
            wspec((1, CP)),
            wspec((CP, CP)),
            wspec((1, CP)),
        ],
        out_specs=pl.BlockSpec((sb, CP), lambda i: (i, 0)),
        out_shape=jax.ShapeDtypeStruct((B * S, CP), _F32),
    )(g, nxp2d, cnt, wx, b1, w2, b2, w3, b3)


def _fold_bn(layer):
    a = layer['gamma'] / jnp.sqrt(layer['rv'] + 1e-5)
    w = layer['W'] * a[:, None]
    b = layer['b'] * a + layer['beta'] - layer['rm'] * a
    return w, b


def kernel(xyz, points, params):
    xs = xyz[:, :, 0]
    ys = xyz[:, :, 1]
    zs = xyz[:, :, 2]
    nx, ny, nz = _fps_call(xs, ys, zs)
    new_xyz = jnp.stack([nx, ny, nz], axis=-1)

    nxp = jnp.concatenate([new_xyz, jnp.zeros((B, S, 5), _F32)], axis=-1)
    xyzt = jnp.transpose(xyz, (0, 2, 1))
    xyztp = jnp.concatenate([xyzt, jnp.zeros((B, 5, N), _F32)], axis=1)
    dist = _dist_call(nxp, xyztp)

    folded = [[_fold_bn(l) for l in params[i]] for i in range(3)]

    def _padc(a):
        return jnp.concatenate(
            [a, jnp.zeros(a.shape[:-1] + (CP - a.shape[-1],), _F32)], -1)

    wps, wxs, b1s = [], [], []
    for i in range(3):
        w1, b1 = folded[i][0]
        wps.append(_padc(jnp.transpose(w1[:, :IN_CH])))
        wx = jnp.transpose(w1[:, IN_CH:])
        wx = jnp.concatenate([wx, jnp.zeros((5, C1S[i]), _F32)], 0)
        wxs.append(_padc(wx))
        b1s.append(_padc(b1[None, :]))

    pts2d = points.reshape(B * N, IN_CH)
    xp2d = jnp.concatenate(
        [xyz.reshape(B * N, 3), jnp.zeros((B * N, 5), _F32)], axis=-1)
    t1, t2, t3 = _p1_call(pts2d, xp2d, wps, wxs)

    g1, g2, g3, cnt = _sc_call(dist.reshape(B * S, N), t1, t2, t3)

    nxp2d = nxp.reshape(B * S, 8)
    outs = []
    for i, g in enumerate((g1, g2, g3)):
        w2, b2 = folded[i][1]
        w3, b3 = folded[i][2]
        w2p = jnp.zeros((CP, CP), _F32).at[:C1S[i], :C2S[i]].set(
            jnp.transpose(w2))
        w3p = jnp.zeros((CP, CP), _F32).at[:C2S[i], :C3S[i]].set(
            jnp.transpose(w3))
        outs.append(_mlp_call(
            i, g, nxp2d, cnt, wxs[i], b1s[i],
            w2p, _padc(b2[None, :]),
            w3p, _padc(b3[None, :])))

    new_points = jnp.concatenate(
        [outs[0][:, :C3S[0]].reshape(B, S, C3S[0]),
         outs[1][:, :C3S[1]].reshape(B, S, C3S[1]),
         outs[2][:, :C3S[2]].reshape(B, S, C3S[2])], axis=-1)
    return (new_xyz, new_points)
